# baseline (speedup 1.0000x reference)
import functools

import jax
import jax.numpy as jnp
from jax import lax
from jax.experimental import pallas as pl
from jax.experimental.pallas import tpu as pltpu
from jax.experimental.pallas import tpu_sc as plsc

N_NODES = 10000
NP = 10240
D = 128
E = 640000
NC, NS = 2, 16
NW = NC * NS
CH = 128
NCHUNK = 160
G = 16
NGRP = NCHUNK // G
R = 2
EPAD = NW * NCHUNK * CH
RPT = NP // NS
AGG_DT = jnp.float32


def _mesh():
    return plsc.VectorSubcoreMesh(core_axis_name="c", subcore_axis_name="s",
                                  num_cores=NC, num_subcores=NS)


@functools.cache
def _make_deg_sc():
    return functools.partial(
        pl.kernel,
        out_type=jax.ShapeDtypeStruct((NC, NP), jnp.float32),
        mesh=_mesh(),
        scratch_types=[
            pltpu.VMEM((NCHUNK, CH), jnp.int32),
            pltpu.VMEM((CH,), jnp.float32),
            pltpu.VMEM_SHARED((NP,), jnp.float32),
        ],
    )(_deg_sc_body)


def _deg_sc_body(dst_hbm, zeros_hbm, out_hbm, dstv, ones_v, acc):
    cid = lax.axis_index("c")
    sid = lax.axis_index("s")
    wid = sid * NC + cid
    pltpu.sync_copy(dst_hbm.at[wid], dstv)
    for b in range(CH // 16):
        ones_v[pl.ds(b * 16, 16)] = jnp.ones((16,), jnp.float32)
    pltpu.sync_copy(zeros_hbm.at[pl.ds(sid * RPT, RPT)],
                    acc.at[pl.ds(sid * RPT, RPT)])
    plsc.subcore_barrier()

    def body(j, carry):
        pltpu.sync_copy(ones_v, acc.at[dstv.at[j]], add=True)
        return carry

    lax.fori_loop(0, NCHUNK, body, 0)
    plsc.subcore_barrier()
    pltpu.sync_copy(acc.at[pl.ds(sid * RPT, RPT)],
                    out_hbm.at[cid, pl.ds(sid * RPT, RPT)])


@functools.cache
def _make_agg_sc():
    return functools.partial(
        pl.kernel,
        out_type=jax.ShapeDtypeStruct((NC, NP, D), AGG_DT),
        mesh=_mesh(),
        scratch_types=[
            pltpu.VMEM((2, G, CH), jnp.int32),
            pltpu.VMEM((2, G, CH), jnp.int32),
            pltpu.VMEM((R, CH, D), AGG_DT),
            pltpu.VMEM_SHARED((NP, D), AGG_DT),
            [pltpu.SemaphoreType.DMA] * R,
            [pltpu.SemaphoreType.DMA] * R,
            pltpu.SemaphoreType.DMA,
        ],
    )(_agg_sc_body)


def _agg_sc_body(src_hbm, dst_hbm, g_hbm, zeros_hbm, out_hbm,
                 sidx, didx, ring, acc, gsem, ssem, isem):
    cid = lax.axis_index("c")
    sid = lax.axis_index("s")
    wid = sid * NC + cid
    pltpu.sync_copy(zeros_hbm.at[pl.ds(sid * RPT, RPT)],
                    acc.at[pl.ds(sid * RPT, RPT)])
    pltpu.make_async_copy(src_hbm.at[wid, pl.ds(0, G)], sidx.at[0],
                          isem).start()
    pltpu.make_async_copy(dst_hbm.at[wid, pl.ds(0, G)], didx.at[0],
                          isem).start()
    plsc.subcore_barrier()

    def grp_body(grp, carry):
        par = lax.rem(grp, 2)
        pltpu.make_async_copy(src_hbm.at[wid, pl.ds(grp * G, G)],
                              sidx.at[par], isem).wait()
        pltpu.make_async_copy(dst_hbm.at[wid, pl.ds(grp * G, G)],
                              didx.at[par], isem).wait()

        @pl.when(grp + 1 < NGRP)
        def _():
            pltpu.make_async_copy(src_hbm.at[wid, pl.ds((grp + 1) * G, G)],
                                  sidx.at[1 - par], isem).start()
            pltpu.make_async_copy(dst_hbm.at[wid, pl.ds((grp + 1) * G, G)],
                                  didx.at[1 - par], isem).start()

        for i in range(R):
            pltpu.make_async_copy(g_hbm.at[sidx.at[par, i]], ring.at[i],
                                  gsem[i]).start()

        def round_body(t, c2):
            base = t * R
            for i in range(R):
                pltpu.make_async_copy(g_hbm.at[sidx.at[par, base + i]],
                                      ring.at[i], gsem[i]).wait()
                pltpu.make_async_copy(ring.at[i],
                                      acc.at[didx.at[par, base + i]],
                                      ssem[i]).start(add=True)
            for i in range(R):
                pltpu.make_async_copy(ring.at[i],
                                      acc.at[didx.at[par, base + i]],
                                      ssem[i]).wait()

                @pl.when(base + i + R < G)
                def _():
                    pltpu.make_async_copy(
                        g_hbm.at[sidx.at[par, base + i + R]], ring.at[i],
                        gsem[i]).start()

            return c2

        lax.fori_loop(0, G // R, round_body, 0)
        return carry

    lax.fori_loop(0, NGRP, grp_body, 0)
    plsc.subcore_barrier()
    pltpu.sync_copy(acc.at[pl.ds(sid * RPT, RPT)],
                    out_hbm.at[cid, pl.ds(sid * RPT, RPT)])


BLK = 1024
GRID = NP // BLK


def _dis(dp_blk):
    deg = dp_blk[0] + dp_blk[1] + 1.0
    return lax.rsqrt(deg)


def _prep1_body(dp_ref, x_ref, w1_ref, g1_ref):
    dis = _dis(dp_ref[...])
    g1_ref[...] = jnp.dot(x_ref[...] * dis[:, None], w1_ref[...],
                          preferred_element_type=jnp.float32).astype(AGG_DT)


def _mid_body(dp_ref, s1_ref, g1_ref, b1_ref, w2_ref, g2_ref):
    dis = _dis(dp_ref[...])
    s = (s1_ref[0].astype(jnp.float32) + s1_ref[1].astype(jnp.float32)
         + g1_ref[...].astype(jnp.float32))
    y = jnp.maximum(dis[:, None] * s + b1_ref[...][None, :], 0.0)
    g2_ref[...] = jnp.dot(y * dis[:, None], w2_ref[...],
                          preferred_element_type=jnp.float32).astype(AGG_DT)


def _final_body(dp_ref, s2_ref, g2_ref, b2_ref, out_ref):
    dis = _dis(dp_ref[...])
    s = (s2_ref[0].astype(jnp.float32) + s2_ref[1].astype(jnp.float32)
         + g2_ref[...].astype(jnp.float32))
    out_ref[...] = dis[:, None] * s + b2_ref[...][None, :]


def _row_spec():
    return pl.BlockSpec((BLK, D), lambda i: (i, 0))


def _dp_spec():
    return pl.BlockSpec((NC, BLK), lambda i: (0, i))


def _s_spec():
    return pl.BlockSpec((NC, BLK, D), lambda i: (0, i, 0))


def _full(shape):
    return pl.BlockSpec(shape, lambda i: tuple(0 for _ in shape))


def _prep1(dp, x, W1):
    return pl.pallas_call(
        _prep1_body,
        grid=(GRID,),
        in_specs=[_dp_spec(), _row_spec(), _full((D, D))],
        out_specs=_row_spec(),
        out_shape=jax.ShapeDtypeStruct((NP, D), AGG_DT),
    )(dp, x, W1)


def _mid(dp, s1, g1, b1, W2):
    return pl.pallas_call(
        _mid_body,
        grid=(GRID,),
        in_specs=[_dp_spec(), _s_spec(), _row_spec(), _full((D,)),
                  _full((D, D))],
        out_specs=_row_spec(),
        out_shape=jax.ShapeDtypeStruct((NP, D), AGG_DT),
    )(dp, s1, g1, b1, W2)


def _final(dp, s2, g2, b2):
    return pl.pallas_call(
        _final_body,
        grid=(GRID,),
        in_specs=[_dp_spec(), _s_spec(), _row_spec(), _full((D,))],
        out_specs=_row_spec(),
        out_shape=jax.ShapeDtypeStruct((NP, D), jnp.float32),
    )(dp, s2, g2, b2)


def kernel(x, edge_index, W1, b1, W2, b2):
    src = edge_index[0].astype(jnp.int32)
    dst = edge_index[1].astype(jnp.int32)
    pad = EPAD - E
    ar = jnp.arange(pad, dtype=jnp.int32)
    src_p = jnp.concatenate([src, ar % N_NODES]).reshape(NW, NCHUNK, CH)
    dst_p = jnp.concatenate([dst, N_NODES + (ar % 16)]).reshape(NW, NCHUNK, CH)

    xp = jnp.zeros((NP, D), jnp.float32).at[:N_NODES].set(x)
    zeros2d = jnp.zeros((NP, D), AGG_DT)
    zeros1d = jnp.zeros((NP,), jnp.float32)

    deg_sc = _make_deg_sc()
    agg_sc = _make_agg_sc()
    dp = deg_sc(dst_p, zeros1d)
    g1 = _prep1(dp, xp, W1)
    s1 = agg_sc(src_p, dst_p, g1, zeros2d)
    g2 = _mid(dp, s1, g1, b1, W2)
    s2 = agg_sc(src_p, dst_p, g2, zeros2d)
    out = _final(dp, s2, g2, b2)
    return out[:N_NODES]


# speedup vs baseline: 1.2471x; 1.2471x over previous
import functools

import jax
import jax.numpy as jnp
from jax import lax
from jax.experimental import pallas as pl
from jax.experimental.pallas import tpu as pltpu
from jax.experimental.pallas import tpu_sc as plsc

N_NODES = 10000
NP = 10240
D = 128
E = 640000
NC, NS = 2, 16
NW = NC * NS
CH = 128
NCHUNK = 160
G = 16
NGRP = NCHUNK // G
R = 2
EPAD = NW * NCHUNK * CH
RPT = NP // NS
AGG_DT = jnp.float32


def _mesh():
    return plsc.VectorSubcoreMesh(core_axis_name="c", subcore_axis_name="s",
                                  num_cores=NC, num_subcores=NS)


@functools.cache
def _make_deg_sc():
    return functools.partial(
        pl.kernel,
        out_type=jax.ShapeDtypeStruct((NC, NP), jnp.float32),
        mesh=_mesh(),
        scratch_types=[
            pltpu.VMEM((NCHUNK, CH), jnp.int32),
            pltpu.VMEM((CH,), jnp.float32),
            pltpu.VMEM_SHARED((NP,), jnp.float32),
            pltpu.SemaphoreType.DMA,
        ],
    )(_deg_sc_body)


_DEG_Q = 8


def _deg_sc_body(dst_hbm, zeros_hbm, out_hbm, dstv, ones_v, acc, dsem):
    cid = lax.axis_index("c")
    sid = lax.axis_index("s")
    wid = sid * NC + cid
    pltpu.sync_copy(dst_hbm.at[wid], dstv)
    for b in range(CH // 16):
        ones_v[pl.ds(b * 16, 16)] = jnp.ones((16,), jnp.float32)
    pltpu.sync_copy(zeros_hbm.at[pl.ds(sid * RPT, RPT)],
                    acc.at[pl.ds(sid * RPT, RPT)])
    plsc.subcore_barrier()

    for j in range(_DEG_Q):
        pltpu.make_async_copy(ones_v, acc.at[dstv.at[j]], dsem).start(add=True)

    def body(j, carry):
        pltpu.make_async_copy(ones_v, acc.at[dstv.at[j]], dsem).wait()
        pltpu.make_async_copy(ones_v, acc.at[dstv.at[j]], dsem).start(add=True)
        return carry

    lax.fori_loop(_DEG_Q, NCHUNK, body, 0)
    for j in range(_DEG_Q):
        pltpu.make_async_copy(ones_v, acc.at[dstv.at[j]], dsem).wait()
    plsc.subcore_barrier()
    pltpu.sync_copy(acc.at[pl.ds(sid * RPT, RPT)],
                    out_hbm.at[cid, pl.ds(sid * RPT, RPT)])


@functools.cache
def _make_agg_sc():
    return functools.partial(
        pl.kernel,
        out_type=jax.ShapeDtypeStruct((NC, NP, D), AGG_DT),
        mesh=_mesh(),
        scratch_types=[
            pltpu.VMEM((2, G, CH), jnp.int32),
            pltpu.VMEM((2, G, CH), jnp.int32),
            pltpu.VMEM((R, CH, D), AGG_DT),
            pltpu.VMEM_SHARED((NP, D), AGG_DT),
            [pltpu.SemaphoreType.DMA] * R,
            [pltpu.SemaphoreType.DMA] * R,
            pltpu.SemaphoreType.DMA,
        ],
    )(_agg_sc_body)


def _agg_sc_body(src_hbm, dst_hbm, g_hbm, zeros_hbm, out_hbm,
                 sidx, didx, ring, acc, gsem, ssem, isem):
    cid = lax.axis_index("c")
    sid = lax.axis_index("s")
    wid = sid * NC + cid
    pltpu.sync_copy(zeros_hbm.at[pl.ds(sid * RPT, RPT)],
                    acc.at[pl.ds(sid * RPT, RPT)])
    pltpu.make_async_copy(src_hbm.at[wid, pl.ds(0, G)], sidx.at[0],
                          isem).start()
    pltpu.make_async_copy(dst_hbm.at[wid, pl.ds(0, G)], didx.at[0],
                          isem).start()
    plsc.subcore_barrier()

    def grp_body(grp, carry):
        par = lax.rem(grp, 2)
        pltpu.make_async_copy(src_hbm.at[wid, pl.ds(grp * G, G)],
                              sidx.at[par], isem).wait()
        pltpu.make_async_copy(dst_hbm.at[wid, pl.ds(grp * G, G)],
                              didx.at[par], isem).wait()

        @pl.when(grp + 1 < NGRP)
        def _():
            pltpu.make_async_copy(src_hbm.at[wid, pl.ds((grp + 1) * G, G)],
                                  sidx.at[1 - par], isem).start()
            pltpu.make_async_copy(dst_hbm.at[wid, pl.ds((grp + 1) * G, G)],
                                  didx.at[1 - par], isem).start()

        ra, rb = ring.at[0], ring.at[1]
        gsa, gsb = gsem[0], gsem[1]
        ssa, ssb = ssem[0], ssem[1]
        pltpu.make_async_copy(g_hbm.at[sidx.at[par, 0]], ra, gsa).start()

        def pair(t, c2):
            t0 = 2 * t
            t1 = t0 + 1

            @pl.when(t > 0)
            def _():
                pltpu.make_async_copy(rb, acc.at[didx.at[par, t1]],
                                      ssb).wait()

            pltpu.make_async_copy(g_hbm.at[sidx.at[par, t1]], rb, gsb).start()
            pltpu.make_async_copy(g_hbm.at[sidx.at[par, t0]], ra, gsa).wait()
            pltpu.make_async_copy(ra, acc.at[didx.at[par, t0]],
                                  ssa).start(add=True)
            pltpu.make_async_copy(g_hbm.at[sidx.at[par, t1]], rb, gsb).wait()
            pltpu.make_async_copy(ra, acc.at[didx.at[par, t0]], ssa).wait()
            pltpu.make_async_copy(rb, acc.at[didx.at[par, t1]],
                                  ssb).start(add=True)

            @pl.when(t + 1 < G // 2)
            def _():
                pltpu.make_async_copy(g_hbm.at[sidx.at[par, t0 + 2]], ra,
                                      gsa).start()

            return c2

        lax.fori_loop(0, G // 2, pair, 0)
        pltpu.make_async_copy(rb, acc.at[didx.at[par, G - 1]], ssb).wait()
        return carry

    lax.fori_loop(0, NGRP, grp_body, 0)
    plsc.subcore_barrier()
    pltpu.sync_copy(acc.at[pl.ds(sid * RPT, RPT)],
                    out_hbm.at[cid, pl.ds(sid * RPT, RPT)])


BLK = 1024
GRID = NP // BLK


def _dis(dp_blk):
    deg = dp_blk[0] + dp_blk[1] + 1.0
    return lax.rsqrt(deg)


def _prep1_body(dp_ref, x_ref, w1_ref, g1_ref):
    dis = _dis(dp_ref[...])
    g1_ref[...] = jnp.dot(x_ref[...] * dis[:, None], w1_ref[...],
                          preferred_element_type=jnp.float32).astype(AGG_DT)


def _mid_body(dp_ref, s1_ref, g1_ref, b1_ref, w2_ref, g2_ref):
    dis = _dis(dp_ref[...])
    s = (s1_ref[0].astype(jnp.float32) + s1_ref[1].astype(jnp.float32)
         + g1_ref[...].astype(jnp.float32))
    y = jnp.maximum(dis[:, None] * s + b1_ref[...][None, :], 0.0)
    g2_ref[...] = jnp.dot(y * dis[:, None], w2_ref[...],
                          preferred_element_type=jnp.float32).astype(AGG_DT)


def _final_body(dp_ref, s2_ref, g2_ref, b2_ref, out_ref):
    dis = _dis(dp_ref[...])
    s = (s2_ref[0].astype(jnp.float32) + s2_ref[1].astype(jnp.float32)
         + g2_ref[...].astype(jnp.float32))
    out_ref[...] = dis[:, None] * s + b2_ref[...][None, :]


def _row_spec():
    return pl.BlockSpec((BLK, D), lambda i: (i, 0))


def _dp_spec():
    return pl.BlockSpec((NC, BLK), lambda i: (0, i))


def _s_spec():
    return pl.BlockSpec((NC, BLK, D), lambda i: (0, i, 0))


def _full(shape):
    return pl.BlockSpec(shape, lambda i: tuple(0 for _ in shape))


def _prep1(dp, x, W1):
    return pl.pallas_call(
        _prep1_body,
        grid=(GRID,),
        in_specs=[_dp_spec(), _row_spec(), _full((D, D))],
        out_specs=_row_spec(),
        out_shape=jax.ShapeDtypeStruct((NP, D), AGG_DT),
    )(dp, x, W1)


def _mid(dp, s1, g1, b1, W2):
    return pl.pallas_call(
        _mid_body,
        grid=(GRID,),
        in_specs=[_dp_spec(), _s_spec(), _row_spec(), _full((D,)),
                  _full((D, D))],
        out_specs=_row_spec(),
        out_shape=jax.ShapeDtypeStruct((NP, D), AGG_DT),
    )(dp, s1, g1, b1, W2)


def _final(dp, s2, g2, b2):
    return pl.pallas_call(
        _final_body,
        grid=(GRID,),
        in_specs=[_dp_spec(), _s_spec(), _row_spec(), _full((D,))],
        out_specs=_row_spec(),
        out_shape=jax.ShapeDtypeStruct((NP, D), jnp.float32),
    )(dp, s2, g2, b2)


def kernel(x, edge_index, W1, b1, W2, b2):
    src = edge_index[0].astype(jnp.int32)
    dst = edge_index[1].astype(jnp.int32)
    pad = EPAD - E
    ar = jnp.arange(pad, dtype=jnp.int32)
    src_p = jnp.concatenate([src, ar % N_NODES]).reshape(NW, NCHUNK, CH)
    dst_p = jnp.concatenate([dst, N_NODES + (ar % 16)]).reshape(NW, NCHUNK, CH)

    xp = jnp.zeros((NP, D), jnp.float32).at[:N_NODES].set(x)
    zeros2d = jnp.zeros((NP, D), AGG_DT)
    zeros1d = jnp.zeros((NP,), jnp.float32)

    deg_sc = _make_deg_sc()
    agg_sc = _make_agg_sc()
    dp = deg_sc(dst_p, zeros1d)
    g1 = _prep1(dp, xp, W1)
    s1 = agg_sc(src_p, dst_p, g1, zeros2d)
    g2 = _mid(dp, s1, g1, b1, W2)
    s2 = agg_sc(src_p, dst_p, g2, zeros2d)
    out = _final(dp, s2, g2, b2)
    return out[:N_NODES]
